# baseline (speedup 1.0000x reference)
import functools

import jax
import jax.numpy as jnp
from jax import lax
from jax.experimental import pallas as pl
from jax.experimental.pallas import tpu as pltpu
from jax.experimental.pallas import tpu_sc as plsc

_B, _G, _D = 32, 8192, 3
_K = 4915
_L = 16
_NCHUNK = _G // _L
_NBIN = 256
_HSTRIDE = _NBIN + 1
_H = _G // 2

_mesh = plsc.VectorSubcoreMesh(core_axis_name="c", subcore_axis_name="s")


@functools.partial(
    pl.kernel,
    mesh=_mesh,
    compiler_params=pltpu.CompilerParams(
        needs_layout_passes=False, use_tc_tiling_on_sc=True
    ),
    out_type=jax.ShapeDtypeStruct((_B, _G), jnp.int32),
    scratch_types=[
        pltpu.VMEM((_G,), jnp.float32),
        pltpu.VMEM((_G,), jnp.float32),
        pltpu.VMEM((_G,), jnp.float32),
        pltpu.VMEM((_B,), jnp.int32),
        pltpu.VMEM((_G,), jnp.int32),
        pltpu.VMEM((_L * _HSTRIDE,), jnp.int32),
        pltpu.VMEM((_G,), jnp.int32),
        pltpu.VMEM((_L,), jnp.float32),
        pltpu.VMEM((_L,), jnp.float32),
        pltpu.VMEM((_L,), jnp.float32),
        pltpu.SemaphoreType.DMA,
        pltpu.SemaphoreType.DMA,
        pltpu.SemaphoreType.DMA,
    ],
)
def _sc_mask(centers_hbm, cidx_hbm, out_hbm, xv, yv, zv, cidx_v, keys_v,
             hist_v, mask_v, qxv, qyv, qzv, sem0, sem1, semq):
    b = lax.axis_index("s") * 2 + lax.axis_index("c")
    h0 = [
        pltpu.async_copy(centers_hbm.at[c, b, pl.ds(0, _H)],
                         r.at[pl.ds(0, _H)], sem0)
        for c, r in enumerate((xv, yv, zv))
    ]
    h1 = [
        pltpu.async_copy(centers_hbm.at[c, b, pl.ds(_H, _H)],
                         r.at[pl.ds(_H, _H)], sem1)
        for c, r in enumerate((xv, yv, zv))
    ]
    pltpu.sync_copy(cidx_hbm, cidx_v)

    lane = lax.iota(jnp.int32, _L)
    zeros = jnp.zeros((_L,), jnp.int32)
    ones = jnp.ones((_L,), jnp.int32)
    lane_base = lane * _HSTRIDE

    bvec = jnp.full((_L,), b, jnp.int32)
    qi = plsc.load_gather(cidx_v, [bvec])
    qbase = (qi[0] // _L) * _L
    qcopies = [
        pltpu.async_copy(centers_hbm.at[c, b, pl.ds(qbase, _L)], r, semq)
        for c, r in enumerate((qxv, qyv, qzv))
    ]

    def zero_hist():
        @plsc.parallel_loop(0, _HSTRIDE, unroll=8)
        def _(i):
            hist_v[pl.ds(i * _L, _L)] = zeros

    zero_hist()

    for c in qcopies:
        c.wait()
    qoff = qi - qbase
    qx = plsc.load_gather(qxv, [qoff])
    qy = plsc.load_gather(qyv, [qoff])
    qz = plsc.load_gather(qzv, [qoff])

    def d2_range(lo, hi):
        @plsc.parallel_loop(lo, hi, unroll=8)
        def _(i):
            x = xv[pl.ds(i * _L, _L)]
            y = yv[pl.ds(i * _L, _L)]
            z = zv[pl.ds(i * _L, _L)]
            dx = x - qx
            dy = y - qy
            dz = z - qz
            d2 = dx * dx + dy * dy + dz * dz
            key = plsc.bitcast(d2, jnp.int32)
            keys_v[pl.ds(i * _L, _L)] = key
            bin_ = lax.shift_right_logical(key, 24)
            plsc.addupdate_scatter(hist_v, [lane_base + bin_], ones)

    for c in h0:
        c.wait()
    d2_range(0, _NCHUNK // 2)
    for c in h1:
        c.wait()
    d2_range(_NCHUNK // 2, _NCHUNK)

    def scan_select(need):
        def chunk(c, carry):
            cum_before, t_acc, ce_acc, cnt_acc = carry
            tot = zeros
            for l in range(_L):
                tot = tot + hist_v[pl.ds(l * _HSTRIDE + c * _L, _L)]
                hist_v[pl.ds(l * _HSTRIDE + c * _L, _L)] = zeros
            cumi = plsc.cumsum(tot) + cum_before
            cume = cumi - tot
            sel = (cume < need) & (cumi >= need)
            bin_ids = c * _L + lane
            t_acc = t_acc + jnp.where(sel, bin_ids, 0)
            ce_acc = ce_acc + jnp.where(sel, cume, 0)
            cnt_acc = cnt_acc + jnp.where(sel, tot, 0)
            cum_before = cum_before + jnp.sum(tot)
            return cum_before, t_acc, ce_acc, cnt_acc

        init = (jnp.int32(0), zeros, zeros, zeros)
        _, t, ce, cnt = lax.fori_loop(0, _NBIN // _L, chunk, init)
        return jnp.sum(t), jnp.sum(ce), jnp.sum(cnt)

    t, ce, cnt = scan_select(jnp.int32(_K))
    prefix = t
    count_less = ce

    for shift in (16, 8, 0):
        @plsc.parallel_loop(0, _NCHUNK, unroll=8)
        def _(i, shift=shift, prefix=prefix):
            key = keys_v[pl.ds(i * _L, _L)]
            active = lax.shift_right_logical(key, shift + 8) == prefix
            bin_ = lax.shift_right_logical(key, shift) & (_NBIN - 1)
            plsc.addupdate_scatter(hist_v, [lane_base + bin_], ones,
                                   mask=active)

        t, ce, cnt = scan_select(jnp.int32(_K) - count_less)
        prefix = (prefix << 8) | t
        count_less = count_less + ce

    kth_key = prefix
    rem = jnp.int32(_K) - count_less

    def write_simple(_):
        @plsc.parallel_loop(0, _NCHUNK, unroll=8)
        def _(i):
            key = keys_v[pl.ds(i * _L, _L)]
            mask_v[pl.ds(i * _L, _L)] = jnp.where(key <= kth_key, 1, 0)

        return jnp.int32(0)

    def write_ties(_):
        def body(i, carry):
            key = keys_v[pl.ds(i * _L, _L)]
            lt = key < kth_key
            eq = key == kth_key
            eqi = jnp.where(eq, 1, 0)
            cum = plsc.cumsum(eqi)
            tie_rank = carry + cum - 1
            inc = lt | (eq & (tie_rank < rem))
            mask_v[pl.ds(i * _L, _L)] = jnp.where(inc, 1, 0)
            return carry + jnp.sum(eqi)

        lax.fori_loop(0, _NCHUNK, body, jnp.int32(0))
        return jnp.int32(0)

    lax.cond(cnt == rem, write_simple, write_ties, jnp.int32(0))

    pltpu.sync_copy(mask_v, out_hbm.at[b])


def kernel(centers, center_idx):
    planes = jnp.transpose(centers, (2, 0, 1))
    return _sc_mask(planes, center_idx).astype(bool)


# speedup vs baseline: 1.0016x; 1.0016x over previous
import functools

import jax
import jax.numpy as jnp
from jax import lax
from jax.experimental import pallas as pl
from jax.experimental.pallas import tpu as pltpu
from jax.experimental.pallas import tpu_sc as plsc

_B, _G, _D = 32, 8192, 3
_K = 4915
_L = 16
_NCHUNK = _G // _L
_NBIN = 256
_HSTRIDE = _NBIN + 1
_H = _G // 2

_mesh = plsc.VectorSubcoreMesh(core_axis_name="c", subcore_axis_name="s")


@functools.partial(
    pl.kernel,
    mesh=_mesh,
    compiler_params=pltpu.CompilerParams(
        needs_layout_passes=False,
        use_tc_tiling_on_sc=True,
        skip_device_barrier=True,
    ),
    out_type=jax.ShapeDtypeStruct((_B, _G), jnp.int32),
    scratch_types=[
        pltpu.VMEM((_G,), jnp.float32),
        pltpu.VMEM((_G,), jnp.float32),
        pltpu.VMEM((_G,), jnp.float32),
        pltpu.VMEM((_B,), jnp.int32),
        pltpu.VMEM((_G,), jnp.int32),
        pltpu.VMEM((_L * _HSTRIDE,), jnp.int32),
        pltpu.VMEM((_G,), jnp.int32),
        pltpu.VMEM((_L,), jnp.float32),
        pltpu.VMEM((_L,), jnp.float32),
        pltpu.VMEM((_L,), jnp.float32),
        pltpu.SemaphoreType.DMA,
        pltpu.SemaphoreType.DMA,
        pltpu.SemaphoreType.DMA,
    ],
)
def _sc_mask(centers_hbm, cidx_hbm, out_hbm, xv, yv, zv, cidx_v, keys_v,
             hist_v, mask_v, qxv, qyv, qzv, sem0, sem1, semq):
    b = lax.axis_index("s") * 2 + lax.axis_index("c")
    h0 = [
        pltpu.async_copy(centers_hbm.at[c, b, pl.ds(0, _H)],
                         r.at[pl.ds(0, _H)], sem0)
        for c, r in enumerate((xv, yv, zv))
    ]
    h1 = [
        pltpu.async_copy(centers_hbm.at[c, b, pl.ds(_H, _H)],
                         r.at[pl.ds(_H, _H)], sem1)
        for c, r in enumerate((xv, yv, zv))
    ]
    pltpu.sync_copy(cidx_hbm, cidx_v)

    lane = lax.iota(jnp.int32, _L)
    zeros = jnp.zeros((_L,), jnp.int32)
    ones = jnp.ones((_L,), jnp.int32)
    lane_base = lane * _HSTRIDE

    bvec = jnp.full((_L,), b, jnp.int32)
    qi = plsc.load_gather(cidx_v, [bvec])
    qbase = (qi[0] // _L) * _L
    qcopies = [
        pltpu.async_copy(centers_hbm.at[c, b, pl.ds(qbase, _L)], r, semq)
        for c, r in enumerate((qxv, qyv, qzv))
    ]

    def zero_hist():
        @plsc.parallel_loop(0, _HSTRIDE, unroll=8)
        def _(i):
            hist_v[pl.ds(i * _L, _L)] = zeros

    zero_hist()

    for c in qcopies:
        c.wait()
    qoff = qi - qbase
    qx = plsc.load_gather(qxv, [qoff])
    qy = plsc.load_gather(qyv, [qoff])
    qz = plsc.load_gather(qzv, [qoff])

    def d2_range(lo, hi):
        @plsc.parallel_loop(lo, hi, unroll=8)
        def _(i):
            x = xv[pl.ds(i * _L, _L)]
            y = yv[pl.ds(i * _L, _L)]
            z = zv[pl.ds(i * _L, _L)]
            dx = x - qx
            dy = y - qy
            dz = z - qz
            d2 = dx * dx + dy * dy + dz * dz
            key = plsc.bitcast(d2, jnp.int32)
            keys_v[pl.ds(i * _L, _L)] = key
            bin_ = lax.shift_right_logical(key, 24)
            plsc.addupdate_scatter(hist_v, [lane_base + bin_], ones)

    for c in h0:
        c.wait()
    d2_range(0, _NCHUNK // 2)
    for c in h1:
        c.wait()
    d2_range(_NCHUNK // 2, _NCHUNK)

    def scan_select(need):
        def chunk(c, carry):
            cum_before, t_acc, ce_acc, cnt_acc = carry
            tot = zeros
            for l in range(_L):
                tot = tot + hist_v[pl.ds(l * _HSTRIDE + c * _L, _L)]
                hist_v[pl.ds(l * _HSTRIDE + c * _L, _L)] = zeros
            cumi = plsc.cumsum(tot) + cum_before
            cume = cumi - tot
            sel = (cume < need) & (cumi >= need)
            bin_ids = c * _L + lane
            t_acc = t_acc + jnp.where(sel, bin_ids, 0)
            ce_acc = ce_acc + jnp.where(sel, cume, 0)
            cnt_acc = cnt_acc + jnp.where(sel, tot, 0)
            cum_before = cum_before + jnp.sum(tot)
            return cum_before, t_acc, ce_acc, cnt_acc

        init = (jnp.int32(0), zeros, zeros, zeros)
        _, t, ce, cnt = lax.fori_loop(0, _NBIN // _L, chunk, init)
        return jnp.sum(t), jnp.sum(ce), jnp.sum(cnt)

    t, ce, cnt = scan_select(jnp.int32(_K))
    prefix = t
    count_less = ce

    for shift in (16, 8, 0):
        @plsc.parallel_loop(0, _NCHUNK, unroll=8)
        def _(i, shift=shift, prefix=prefix):
            key = keys_v[pl.ds(i * _L, _L)]
            active = lax.shift_right_logical(key, shift + 8) == prefix
            bin_ = lax.shift_right_logical(key, shift) & (_NBIN - 1)
            plsc.addupdate_scatter(hist_v, [lane_base + bin_], ones,
                                   mask=active)

        t, ce, cnt = scan_select(jnp.int32(_K) - count_less)
        prefix = (prefix << 8) | t
        count_less = count_less + ce

    kth_key = prefix
    rem = jnp.int32(_K) - count_less

    def write_simple(_):
        @plsc.parallel_loop(0, _NCHUNK, unroll=8)
        def _(i):
            key = keys_v[pl.ds(i * _L, _L)]
            mask_v[pl.ds(i * _L, _L)] = jnp.where(key <= kth_key, 1, 0)

        return jnp.int32(0)

    def write_ties(_):
        def body(i, carry):
            key = keys_v[pl.ds(i * _L, _L)]
            lt = key < kth_key
            eq = key == kth_key
            eqi = jnp.where(eq, 1, 0)
            cum = plsc.cumsum(eqi)
            tie_rank = carry + cum - 1
            inc = lt | (eq & (tie_rank < rem))
            mask_v[pl.ds(i * _L, _L)] = jnp.where(inc, 1, 0)
            return carry + jnp.sum(eqi)

        lax.fori_loop(0, _NCHUNK, body, jnp.int32(0))
        return jnp.int32(0)

    lax.cond(cnt == rem, write_simple, write_ties, jnp.int32(0))

    pltpu.sync_copy(mask_v, out_hbm.at[b])


def kernel(centers, center_idx):
    planes = jnp.transpose(centers, (2, 0, 1))
    return _sc_mask(planes, center_idx).astype(bool)
